# Initial kernel scaffold; baseline (speedup 1.0000x reference)
import jax
import jax.numpy as jnp
from jax.experimental import pallas as pl


def kernel(V_rec, V_gt, faces, L_indices, L_values):
    raise NotImplementedError("write your pallas kernel here")


# speedup vs baseline: 8.6374x; 8.6374x over previous
import jax
import jax.numpy as jnp
from jax import lax
from jax.experimental import pallas as pl
from jax.experimental.pallas import tpu as pltpu
from jax.experimental.pallas import tpu_sc as plsc

NV = 50000
NF = 100000
NNZ = 350000

NC = 2
NS = 16
NW = NC * NS

TS = 3136
VP = NS * TS

CH = 25
CW = CH * 128
FPAD = NW * CW
LSEG = 4
LPAD = NW * LSEG * CW

_f32 = jnp.float32
_i32 = jnp.int32


def _sc_body(txg, tyg, tzg, txr, tyr, tzr,
             fi0, fi1, fi2, lrow, lcol, lval,
             out,
             T0, T1, T2, T3, T4, T5,
             A0, A1, A2, A3, A4, A5, A6, A7, A8, A9, A10, A11,
             I0, I1, I2,
             D0, D1, D2, D3, D4, D5, D6, D7, D8,
             ZB):
    c = lax.axis_index("c")
    s = lax.axis_index("s")
    w = c * NS + s

    tbl = (T0, T1, T2, T3, T4, T5)
    ins = (txg, tyg, tzg, txr, tyr, tzr)
    accs = (A0, A1, A2, A3, A4, A5, A6, A7, A8, A9, A10, A11)
    Is = (I0, I1, I2)
    Ds = (D0, D1, D2, D3, D4, D5, D6, D7, D8)

    sbase = pl.multiple_of(s * TS, 8)

    for k in range(6):
        pltpu.sync_copy(ins[k].at[pl.ds(sbase, TS)], ZB)
        pltpu.sync_copy(ZB, tbl[k].at[pl.ds(sbase, TS)])

    def _zb(i, carry):
        ZB[pl.ds(pl.multiple_of(i * 16, 16), 16)] = jnp.zeros((16,), _f32)
        return carry

    lax.fori_loop(0, TS // 16, _zb, 0)
    for k in range(12):
        pltpu.sync_copy(ZB, accs[k].at[pl.ds(sbase, TS)])
    plsc.subcore_barrier()

    for v in range(3):
        pltpu.sync_copy((fi0, fi1, fi2)[v].at[pl.ds(w * CH, CH)], Is[v])

    for m in range(2):
        tb = 3 * m
        for v in range(3):
            for k in range(3):
                def _g(j, carry, v=v, k=k, tb=tb):
                    pltpu.sync_copy(tbl[tb + k].at[Is[v].at[j]],
                                    Ds[3 * v + k].at[j])
                    return carry
                lax.fori_loop(0, CH, _g, 0)

        def _cross_row(j, carry):
            def _cross(kk, carry2):
                sl = pl.ds(pl.multiple_of(kk * 16, 16), 16)
                x0 = D0[j, sl]; y0 = D1[j, sl]; z0 = D2[j, sl]
                x1 = D3[j, sl]; y1 = D4[j, sl]; z1 = D5[j, sl]
                x2 = D6[j, sl]; y2 = D7[j, sl]; z2 = D8[j, sl]
                e1x = x1 - x0; e1y = y1 - y0; e1z = z1 - z0
                e2x = x2 - x0; e2y = y2 - y0; e2z = z2 - z0
                D0[j, sl] = e1y * e2z - e1z * e2y
                D1[j, sl] = e1z * e2x - e1x * e2z
                D2[j, sl] = e1x * e2y - e1y * e2x
                return carry2
            return lax.fori_loop(0, 8, _cross, carry)

        lax.fori_loop(0, CH, _cross_row, 0)

        for v in range(3):
            for k in range(3):
                def _sa(j, carry, v=v, k=k, tb=tb):
                    pltpu.sync_copy(Ds[k].at[j], accs[tb + k].at[Is[v].at[j]],
                                    add=True)
                    return carry
                lax.fori_loop(0, CH, _sa, 0)

    for seg in range(LSEG):
        cb = (w * LSEG + seg) * CH
        pltpu.sync_copy(lcol.at[pl.ds(cb, CH)], I0)
        pltpu.sync_copy(lrow.at[pl.ds(cb, CH)], I1)
        pltpu.sync_copy(lval.at[pl.ds(cb, CH)], D3)
        for m in range(2):
            tb = 3 * m
            ab = 6 + 3 * m
            for k in range(3):
                def _lg(j, carry, k=k, tb=tb):
                    pltpu.sync_copy(tbl[tb + k].at[I0.at[j]], Ds[k].at[j])
                    return carry
                lax.fori_loop(0, CH, _lg, 0)

            def _mul_row(j, carry):
                def _mul(kk, carry2):
                    sl = pl.ds(pl.multiple_of(kk * 16, 16), 16)
                    vv = D3[j, sl]
                    D0[j, sl] = D0[j, sl] * vv
                    D1[j, sl] = D1[j, sl] * vv
                    D2[j, sl] = D2[j, sl] * vv
                    return carry2
                return lax.fori_loop(0, 8, _mul, carry)

            lax.fori_loop(0, CH, _mul_row, 0)

            for k in range(3):
                def _lsa(j, carry, k=k, ab=ab):
                    pltpu.sync_copy(Ds[k].at[j], accs[ab + k].at[I1.at[j]],
                                    add=True)
                    return carry
                lax.fori_loop(0, CH, _lsa, 0)

    plsc.subcore_barrier()
    for q in range(12):
        pltpu.sync_copy(accs[q].at[pl.ds(sbase, TS)], ZB)
        pltpu.sync_copy(ZB, out.at[q * 2 + c, pl.ds(sbase, TS)])


_SC_SCRATCH = (
    [pltpu.VMEM_SHARED((VP,), _f32)] * 6
    + [pltpu.VMEM_SHARED((VP,), _f32)] * 12
    + [pltpu.VMEM((CH, 128), _i32)] * 3
    + [pltpu.VMEM((CH, 128), _f32)] * 9
    + [pltpu.VMEM((TS,), _f32)]
)

_sc_kernel = pl.kernel(
    _sc_body,
    out_type=jax.ShapeDtypeStruct((24, VP), _f32),
    mesh=plsc.VectorSubcoreMesh(core_axis_name="c", subcore_axis_name="s"),
    scratch_types=_SC_SCRATCH,
    compiler_params=pltpu.CompilerParams(use_tc_tiling_on_sc=False),
)


def _tc_body(p_ref, vr_ref, vg_ref, o_ref):
    def acc(q):
        return p_ref[2 * q:2 * q + 1, :] + p_ref[2 * q + 1:2 * q + 2, :]

    nxg, nyg, nzg = acc(0), acc(1), acc(2)
    nxr, nyr, nzr = acc(3), acc(4), acc(5)
    dxg, dyg, dzg = acc(6), acc(7), acc(8)
    dxr, dyr, dzr = acc(9), acc(10), acc(11)

    mask = lax.broadcasted_iota(_i32, (1, VP), 1) < NV
    inv_nv = _f32(1.0 / NV)

    na = jnp.sqrt(nxg * nxg + nyg * nyg + nzg * nzg)
    nb = jnp.sqrt(nxr * nxr + nyr * nyr + nzr * nzr)
    ma = jnp.maximum(na, _f32(1e-6))
    mb = jnp.maximum(nb, _f32(1e-6))
    dot_n = (nxg * nxr + nyg * nyr + nzg * nzr) / (ma * mb)
    den_n = jnp.maximum((na / ma) * (nb / mb), _f32(1e-8))
    cos_n = dot_n / den_n
    cos_n = jnp.where(cos_n == cos_n, cos_n, _f32(1.0))
    loss_normal = jnp.sum(jnp.where(mask, _f32(1.0) - cos_n, _f32(0.0))) * inv_nv

    la = jnp.sqrt(dxg * dxg + dyg * dyg + dzg * dzg)
    lb = jnp.sqrt(dxr * dxr + dyr * dyr + dzr * dzr)
    dot_l = dxg * dxr + dyg * dyr + dzg * dzr
    cos_l = dot_l / jnp.maximum(la * lb, _f32(1e-8))
    cos_l = jnp.where(cos_l == cos_l, cos_l, _f32(1.0))
    loss_lap = jnp.sum(jnp.where(mask, _f32(1.0) - cos_l, _f32(0.0))) * inv_nv
    mx = _f32(0.0)
    for d in (dxg, dyg, dzg, dxr, dyr, dzr):
        mx = jnp.maximum(mx, jnp.max(jnp.where(mask, jnp.abs(d), _f32(0.0))))
    finite = jnp.logical_and(mx == mx, mx < jnp.inf)
    loss_lap = jnp.where(finite, loss_lap, _f32(0.0))

    loss_l1 = jnp.sum(jnp.abs(vr_ref[...] - vg_ref[...])) * _f32(1.0 / (NV * 3))

    total = loss_l1 + loss_normal + _f32(0.1) * loss_lap

    ri = lax.broadcasted_iota(_i32, (8, 128), 0)
    li = lax.broadcasted_iota(_i32, (8, 128), 1)
    row0 = jnp.where(li == 0, total,
                     jnp.where(li == 1, loss_l1,
                               jnp.where(li == 2, loss_normal,
                                         jnp.where(li == 3, loss_lap,
                                                   _f32(0.0)))))
    o_ref[...] = jnp.where(ri == 0, row0, _f32(0.0))


_tc_kernel = pl.pallas_call(
    _tc_body,
    out_shape=jax.ShapeDtypeStruct((8, 128), _f32),
)

_L1P = 1172 * 128


@jax.jit
def kernel(V_rec, V_gt, faces, L_indices, L_values):
    pv = VP - NV
    cols = [jnp.pad(V_gt[:, i], (0, pv)) for i in range(3)]
    cols += [jnp.pad(V_rec[:, i], (0, pv)) for i in range(3)]
    fis = [jnp.pad(faces[:, i], (0, FPAD - NF)).reshape(NW * CH, 128)
           for i in range(3)]
    lr = jnp.pad(L_indices[0], (0, LPAD - NNZ)).reshape(-1, 128)
    lc = jnp.pad(L_indices[1], (0, LPAD - NNZ)).reshape(-1, 128)
    lv = jnp.pad(L_values, (0, LPAD - NNZ)).reshape(-1, 128)

    part = _sc_kernel(*cols, *fis, lr, lc, lv)

    vr2 = jnp.pad(V_rec.reshape(-1), (0, _L1P - NV * 3)).reshape(1172, 128)
    vg2 = jnp.pad(V_gt.reshape(-1), (0, _L1P - NV * 3)).reshape(1172, 128)
    o = _tc_kernel(part, vr2, vg2)
    return (o[0, 0], o[0, 1], o[0, 2], o[0, 3])


# speedup vs baseline: 8.7846x; 1.0170x over previous
import jax
import jax.numpy as jnp
from jax import lax
from jax.experimental import pallas as pl
from jax.experimental.pallas import tpu as pltpu
from jax.experimental.pallas import tpu_sc as plsc

NV = 50000
NF = 100000
NNZ = 350000

NC = 2
NS = 16
NW = NC * NS

TS = 3200
VP = NS * TS
HOP = 1600

FCH = 26
FPAD = NW * FCH * 128
LCH = 100
LGRP = 4
LCP = LCH // LGRP
LPAD = NW * LCH * 128

_f32 = jnp.float32
_i32 = jnp.int32


def _sc_body(tab, fi0, fi1, fi2, lcol, lrow, lval, out, *scr):
    T = scr[0:6]
    AN = scr[6:12]
    AL = scr[12:18]
    I0, I1, I2 = scr[18:21]
    A = scr[21:27]
    B = scr[27:33]
    C = scr[33:39]
    N = scr[39:45]
    VB = scr[45]
    S1 = scr[46]

    c = lax.axis_index("c")
    s = lax.axis_index("s")
    w = c * NS + s
    sbase = pl.multiple_of(s * TS, 8)

    z16 = jnp.zeros((16,), _f32)
    for i in range(HOP // 16):
        S1[pl.ds(i * 16, 16)] = z16
    for h in range(2):
        rb = pl.multiple_of(sbase + h * HOP, 8)
        for q in range(6):
            pltpu.sync_copy(S1, AN[q].at[pl.ds(rb, HOP)])
            pltpu.sync_copy(S1, AL[q].at[pl.ds(rb, HOP)])

    for h in range(2):
        rb = pl.multiple_of(sbase + h * HOP, 8)
        for p in range(6):
            pltpu.sync_copy(tab.at[p, pl.ds(rb, HOP)], S1)
            pltpu.sync_copy(S1, T[p].at[pl.ds(rb, HOP)])
    plsc.subcore_barrier()

    pltpu.sync_copy(fi0.at[pl.ds(w * FCH, FCH)], I0)
    pltpu.sync_copy(fi1.at[pl.ds(w * FCH, FCH)], I1)
    pltpu.sync_copy(fi2.at[pl.ds(w * FCH, FCH)], I2)

    def _face_chunk(j, carry):
        i0 = I0.at[j]
        i1 = I1.at[j]
        i2 = I2.at[j]
        for p in range(6):
            pltpu.sync_copy(T[p].at[i0], A[p])
            pltpu.sync_copy(T[p].at[i1], B[p])
            pltpu.sync_copy(T[p].at[i2], C[p])
        for v in range(8):
            sl = pl.ds(v * 16, 16)
            for m in (0, 3):
                e1 = [B[m + d][sl] - A[m + d][sl] for d in range(3)]
                e2 = [C[m + d][sl] - A[m + d][sl] for d in range(3)]
                N[m + 0][sl] = e1[1] * e2[2] - e1[2] * e2[1]
                N[m + 1][sl] = e1[2] * e2[0] - e1[0] * e2[2]
                N[m + 2][sl] = e1[0] * e2[1] - e1[1] * e2[0]
        for p in range(6):
            pltpu.sync_copy(N[p], AN[p].at[i0], add=True)
            pltpu.sync_copy(N[p], AN[p].at[i1], add=True)
            pltpu.sync_copy(N[p], AN[p].at[i2], add=True)
        return carry

    lax.fori_loop(0, FCH, _face_chunk, 0)

    for g in range(LGRP):
        gb = w * LCH + g * LCP
        pltpu.sync_copy(lcol.at[pl.ds(gb, LCP)], I0.at[pl.ds(0, LCP)])
        pltpu.sync_copy(lrow.at[pl.ds(gb, LCP)], I1.at[pl.ds(0, LCP)])

        def _lap_chunk(j, carry, gb=gb):
            ic = I0.at[j]
            ir = I1.at[j]
            for p in range(6):
                pltpu.sync_copy(T[p].at[ic], A[p])
            pltpu.sync_copy(
                lval.at[pl.ds(pl.multiple_of((gb + j) * 128, 128), 128)], VB)
            for v in range(8):
                sl = pl.ds(v * 16, 16)
                vv = VB[sl]
                for p in range(6):
                    N[p][sl] = A[p][sl] * vv
            for p in range(6):
                pltpu.sync_copy(N[p], AL[p].at[ir], add=True)
            return carry

        lax.fori_loop(0, LCP, _lap_chunk, 0)

    plsc.subcore_barrier()
    accs = AN + AL
    for q in range(12):
        for h in range(2):
            rb = pl.multiple_of(sbase + h * HOP, 8)
            pltpu.sync_copy(accs[q].at[pl.ds(rb, HOP)], S1)
            pltpu.sync_copy(S1, out.at[q * 2 + c, pl.ds(rb, HOP)])


_SC_SCRATCH = (
    [pltpu.VMEM_SHARED((VP,), _f32)] * 18
    + [pltpu.VMEM((FCH, 128), _i32)] * 3
    + [pltpu.VMEM((128,), _f32)] * 25
    + [pltpu.VMEM((HOP,), _f32)]
)

_sc_kernel = pl.kernel(
    _sc_body,
    out_type=jax.ShapeDtypeStruct((24, VP), _f32),
    mesh=plsc.VectorSubcoreMesh(core_axis_name="c", subcore_axis_name="s"),
    scratch_types=_SC_SCRATCH,
    compiler_params=pltpu.CompilerParams(use_tc_tiling_on_sc=False),
)


def _tc_body(p_ref, vr_ref, vg_ref, o_ref):
    def acc(q):
        return p_ref[2 * q:2 * q + 1, :] + p_ref[2 * q + 1:2 * q + 2, :]

    nxg, nyg, nzg = acc(0), acc(1), acc(2)
    nxr, nyr, nzr = acc(3), acc(4), acc(5)
    dxg, dyg, dzg = acc(6), acc(7), acc(8)
    dxr, dyr, dzr = acc(9), acc(10), acc(11)

    mask = lax.broadcasted_iota(_i32, (1, VP), 1) < NV
    inv_nv = _f32(1.0 / NV)

    na = jnp.sqrt(nxg * nxg + nyg * nyg + nzg * nzg)
    nb = jnp.sqrt(nxr * nxr + nyr * nyr + nzr * nzr)
    ma = jnp.maximum(na, _f32(1e-6))
    mb = jnp.maximum(nb, _f32(1e-6))
    dot_n = (nxg * nxr + nyg * nyr + nzg * nzr) / (ma * mb)
    den_n = jnp.maximum((na / ma) * (nb / mb), _f32(1e-8))
    cos_n = dot_n / den_n
    cos_n = jnp.where(cos_n == cos_n, cos_n, _f32(1.0))
    loss_normal = jnp.sum(jnp.where(mask, _f32(1.0) - cos_n, _f32(0.0))) * inv_nv

    la = jnp.sqrt(dxg * dxg + dyg * dyg + dzg * dzg)
    lb = jnp.sqrt(dxr * dxr + dyr * dyr + dzr * dzr)
    dot_l = dxg * dxr + dyg * dyr + dzg * dzr
    cos_l = dot_l / jnp.maximum(la * lb, _f32(1e-8))
    cos_l = jnp.where(cos_l == cos_l, cos_l, _f32(1.0))
    loss_lap = jnp.sum(jnp.where(mask, _f32(1.0) - cos_l, _f32(0.0))) * inv_nv
    mx = _f32(0.0)
    for d in (dxg, dyg, dzg, dxr, dyr, dzr):
        mx = jnp.maximum(mx, jnp.max(jnp.where(mask, jnp.abs(d), _f32(0.0))))
    finite = jnp.logical_and(mx == mx, mx < jnp.inf)
    loss_lap = jnp.where(finite, loss_lap, _f32(0.0))

    loss_l1 = jnp.sum(jnp.abs(vr_ref[...] - vg_ref[...])) * _f32(1.0 / (NV * 3))

    total = loss_l1 + loss_normal + _f32(0.1) * loss_lap

    ri = lax.broadcasted_iota(_i32, (8, 128), 0)
    li = lax.broadcasted_iota(_i32, (8, 128), 1)
    row0 = jnp.where(li == 0, total,
                     jnp.where(li == 1, loss_l1,
                               jnp.where(li == 2, loss_normal,
                                         jnp.where(li == 3, loss_lap,
                                                   _f32(0.0)))))
    o_ref[...] = jnp.where(ri == 0, row0, _f32(0.0))


_tc_kernel = pl.pallas_call(
    _tc_body,
    out_shape=jax.ShapeDtypeStruct((8, 128), _f32),
)

_L1P = 1172 * 128


@jax.jit
def kernel(V_rec, V_gt, faces, L_indices, L_values):
    tab = jnp.concatenate([V_gt.T, V_rec.T], axis=0)
    tab = jnp.pad(tab, ((0, 0), (0, VP - NV)))
    fis = [jnp.pad(faces[:, i], (0, FPAD - NF)).reshape(NW * FCH, 128)
           for i in range(3)]
    lr = jnp.pad(L_indices[0], (0, LPAD - NNZ)).reshape(-1, 128)
    lc = jnp.pad(L_indices[1], (0, LPAD - NNZ)).reshape(-1, 128)
    lv = jnp.pad(L_values, (0, LPAD - NNZ))

    part = _sc_kernel(tab, *fis, lc, lr, lv)

    vr2 = jnp.pad(V_rec.reshape(-1), (0, _L1P - NV * 3)).reshape(1172, 128)
    vg2 = jnp.pad(V_gt.reshape(-1), (0, _L1P - NV * 3)).reshape(1172, 128)
    o = _tc_kernel(part, vr2, vg2)
    return (o[0, 0], o[0, 1], o[0, 2], o[0, 3])


# speedup vs baseline: 9.3943x; 1.0694x over previous
import jax
import jax.numpy as jnp
from jax import lax
from jax.experimental import pallas as pl
from jax.experimental.pallas import tpu as pltpu
from jax.experimental.pallas import tpu_sc as plsc

NV = 50000
NF = 100000
NNZ = 350000

NC = 2
NS = 16
NW = NC * NS

TS = 3200
VP = NS * TS
HOP = 3200

FCH = 25
FPAD = NW * FCH * 128
LCH = 100
LGRP = 4
LCP = LCH // LGRP
LPAD = NW * LCH * 128

_f32 = jnp.float32
_i32 = jnp.int32


def _sc_body(tab, fi0, fi1, fi2, lcol, lrow, lval, out, *scr):
    T = scr[0:6]
    AN = scr[6:12]
    AL = scr[12:18]
    I0, I1, I2 = scr[18:21]
    A = scr[21:27]
    B = scr[27:33]
    C = scr[33:39]
    N = scr[39:45]
    VB = scr[45]
    S1 = scr[46]

    c = lax.axis_index("c")
    s = lax.axis_index("s")
    w = c * NS + s
    sbase = pl.multiple_of(s * TS, 8)

    z16 = jnp.zeros((16,), _f32)
    for i in range(HOP // 16):
        S1[pl.ds(i * 16, 16)] = z16
    for h in range(1):
        rb = pl.multiple_of(sbase + h * HOP, 8)
        for q in range(6):
            pltpu.sync_copy(S1, AN[q].at[pl.ds(rb, HOP)])
            pltpu.sync_copy(S1, AL[q].at[pl.ds(rb, HOP)])

    for h in range(1):
        rb = pl.multiple_of(sbase + h * HOP, 8)
        for p in range(6):
            pltpu.sync_copy(tab.at[p, pl.ds(rb, HOP)], S1)
            pltpu.sync_copy(S1, T[p].at[pl.ds(rb, HOP)])
    plsc.subcore_barrier()

    pltpu.sync_copy(fi0.at[pl.ds(w * FCH, FCH)], I0)
    pltpu.sync_copy(fi1.at[pl.ds(w * FCH, FCH)], I1)
    pltpu.sync_copy(fi2.at[pl.ds(w * FCH, FCH)], I2)

    def _face_chunk(j, carry):
        i0 = I0.at[j]
        i1 = I1.at[j]
        i2 = I2.at[j]
        for p in range(6):
            pltpu.sync_copy(T[p].at[i0], A[p])
            pltpu.sync_copy(T[p].at[i1], B[p])
            pltpu.sync_copy(T[p].at[i2], C[p])
        for v in range(8):
            sl = pl.ds(v * 16, 16)
            for m in (0, 3):
                e1 = [B[m + d][sl] - A[m + d][sl] for d in range(3)]
                e2 = [C[m + d][sl] - A[m + d][sl] for d in range(3)]
                N[m + 0][sl] = e1[1] * e2[2] - e1[2] * e2[1]
                N[m + 1][sl] = e1[2] * e2[0] - e1[0] * e2[2]
                N[m + 2][sl] = e1[0] * e2[1] - e1[1] * e2[0]
        for p in range(6):
            pltpu.sync_copy(N[p], AN[p].at[i0], add=True)
            pltpu.sync_copy(N[p], AN[p].at[i1], add=True)
            pltpu.sync_copy(N[p], AN[p].at[i2], add=True)
        return carry

    lax.fori_loop(0, FCH, _face_chunk, 0)

    for g in range(LGRP):
        gb = w * LCH + g * LCP
        pltpu.sync_copy(lcol.at[pl.ds(gb, LCP)], I0.at[pl.ds(0, LCP)])
        pltpu.sync_copy(lrow.at[pl.ds(gb, LCP)], I1.at[pl.ds(0, LCP)])

        def _lap_chunk(j, carry, gb=gb):
            ic = I0.at[j]
            ir = I1.at[j]
            for p in range(6):
                pltpu.sync_copy(T[p].at[ic], A[p])
            pltpu.sync_copy(
                lval.at[pl.ds(pl.multiple_of((gb + j) * 128, 128), 128)], VB)
            for v in range(8):
                sl = pl.ds(v * 16, 16)
                vv = VB[sl]
                for p in range(6):
                    N[p][sl] = A[p][sl] * vv
            for p in range(6):
                pltpu.sync_copy(N[p], AL[p].at[ir], add=True)
            return carry

        lax.fori_loop(0, LCP, _lap_chunk, 0)

    plsc.subcore_barrier()
    accs = AN + AL
    for q in range(12):
        for h in range(1):
            rb = pl.multiple_of(sbase + h * HOP, 8)
            pltpu.sync_copy(accs[q].at[pl.ds(rb, HOP)], S1)
            pltpu.sync_copy(S1, out.at[q * 2 + c, pl.ds(rb, HOP)])


_SC_SCRATCH = (
    [pltpu.VMEM_SHARED((VP,), _f32)] * 18
    + [pltpu.VMEM((FCH, 128), _i32)] * 3
    + [pltpu.VMEM((128,), _f32)] * 25
    + [pltpu.VMEM((HOP,), _f32)]
)

_sc_kernel = pl.kernel(
    _sc_body,
    out_type=jax.ShapeDtypeStruct((24, VP), _f32),
    mesh=plsc.VectorSubcoreMesh(core_axis_name="c", subcore_axis_name="s"),
    scratch_types=_SC_SCRATCH,
    compiler_params=pltpu.CompilerParams(use_tc_tiling_on_sc=False),
)


def _tc_body(p_ref, vr_ref, vg_ref, o_ref):
    def acc(q):
        return p_ref[2 * q:2 * q + 1, :] + p_ref[2 * q + 1:2 * q + 2, :]

    nxg, nyg, nzg = acc(0), acc(1), acc(2)
    nxr, nyr, nzr = acc(3), acc(4), acc(5)
    dxg, dyg, dzg = acc(6), acc(7), acc(8)
    dxr, dyr, dzr = acc(9), acc(10), acc(11)

    mask = lax.broadcasted_iota(_i32, (1, VP), 1) < NV
    inv_nv = _f32(1.0 / NV)

    na = jnp.sqrt(nxg * nxg + nyg * nyg + nzg * nzg)
    nb = jnp.sqrt(nxr * nxr + nyr * nyr + nzr * nzr)
    ma = jnp.maximum(na, _f32(1e-6))
    mb = jnp.maximum(nb, _f32(1e-6))
    dot_n = (nxg * nxr + nyg * nyr + nzg * nzr) / (ma * mb)
    den_n = jnp.maximum((na / ma) * (nb / mb), _f32(1e-8))
    cos_n = dot_n / den_n
    cos_n = jnp.where(cos_n == cos_n, cos_n, _f32(1.0))
    loss_normal = jnp.sum(jnp.where(mask, _f32(1.0) - cos_n, _f32(0.0))) * inv_nv

    la = jnp.sqrt(dxg * dxg + dyg * dyg + dzg * dzg)
    lb = jnp.sqrt(dxr * dxr + dyr * dyr + dzr * dzr)
    dot_l = dxg * dxr + dyg * dyr + dzg * dzr
    cos_l = dot_l / jnp.maximum(la * lb, _f32(1e-8))
    cos_l = jnp.where(cos_l == cos_l, cos_l, _f32(1.0))
    loss_lap = jnp.sum(jnp.where(mask, _f32(1.0) - cos_l, _f32(0.0))) * inv_nv
    mx = _f32(0.0)
    for d in (dxg, dyg, dzg, dxr, dyr, dzr):
        mx = jnp.maximum(mx, jnp.max(jnp.where(mask, jnp.abs(d), _f32(0.0))))
    finite = jnp.logical_and(mx == mx, mx < jnp.inf)
    loss_lap = jnp.where(finite, loss_lap, _f32(0.0))

    loss_l1 = jnp.sum(jnp.abs(vr_ref[...] - vg_ref[...])) * _f32(1.0 / (NV * 3))

    total = loss_l1 + loss_normal + _f32(0.1) * loss_lap

    ri = lax.broadcasted_iota(_i32, (8, 128), 0)
    li = lax.broadcasted_iota(_i32, (8, 128), 1)
    row0 = jnp.where(li == 0, total,
                     jnp.where(li == 1, loss_l1,
                               jnp.where(li == 2, loss_normal,
                                         jnp.where(li == 3, loss_lap,
                                                   _f32(0.0)))))
    o_ref[...] = jnp.where(ri == 0, row0, _f32(0.0))


_tc_kernel = pl.pallas_call(
    _tc_body,
    out_shape=jax.ShapeDtypeStruct((8, 128), _f32),
)

_L1P = 1172 * 128


@jax.jit
def kernel(V_rec, V_gt, faces, L_indices, L_values):
    tab = jnp.concatenate([V_gt.T, V_rec.T], axis=0)
    tab = jnp.pad(tab, ((0, 0), (0, VP - NV)))
    fis = [jnp.pad(faces[:, i], (0, FPAD - NF)).reshape(NW * FCH, 128)
           for i in range(3)]
    lr = jnp.pad(L_indices[0], (0, LPAD - NNZ)).reshape(-1, 128)
    lc = jnp.pad(L_indices[1], (0, LPAD - NNZ)).reshape(-1, 128)
    lv = jnp.pad(L_values, (0, LPAD - NNZ))

    part = _sc_kernel(tab, *fis, lc, lr, lv)

    vr2 = jnp.pad(V_rec.reshape(-1), (0, _L1P - NV * 3)).reshape(1172, 128)
    vg2 = jnp.pad(V_gt.reshape(-1), (0, _L1P - NV * 3)).reshape(1172, 128)
    o = _tc_kernel(part, vr2, vg2)
    return (o[0, 0], o[0, 1], o[0, 2], o[0, 3])
